# baseline (device time: 92347 ns/iter reference)
import functools

import jax
import jax.numpy as jnp
from jax import lax
from jax.experimental import pallas as pl
from jax.experimental.pallas import tpu as pltpu

N_DEV = 4
SQ = 2048
SKV = 2048
DM = 1024
HL = 8
DH = 128
DL = HL * DH
QB = 256
KW = 512
WIN = 128
QROWS = SQ // N_DEV
SCALE = 0.08838834764831843
QSCALE = SCALE * 1.4426950408889634
HQB = 128


def kernel(x, Wq, K_ext, V_ext, Wo):
    xb = x[0]
    kb = K_ext[0]
    vb = V_ext[0]

    def body(x_hbm, wq_hbm, k_hbm, v_hbm, wo_hbm, out_ref,
             stagA, stagB, x_ref, k_ref, v_ref, wq_ref, wo_ref,
             q_ref, ctx_ref,
             dr_send, dr_recv, ag_sendR, ag_recvR, ag_sendL, ag_recvL,
             copyA_sems, copyB_sems, drs_sems, drr_sems,
             sendR_sems, recvR_sems, sendL_sems, recvL_sems):
        my_pos = lax.axis_index("i")
        left = lax.rem(my_pos + N_DEV - 1, N_DEV)
        right = lax.rem(my_pos + 1, N_DEV)
        diag = lax.rem(my_pos + 2, N_DEV)

        barrier_sem = pltpu.get_barrier_semaphore()
        for nbr in (left, right, diag):
            pl.semaphore_signal(barrier_sem, inc=1, device_id=(nbr,),
                                device_id_type=pl.DeviceIdType.MESH)

        CH = 1024
        srcsA = [
            wq_hbm.at[:, pl.ds(my_pos * DL, DL)],
            x_hbm.at[pl.ds(0, CH), :],
            x_hbm.at[pl.ds(CH, CH), :],
            wo_hbm.at[pl.ds(my_pos * DL, DL), :],
        ]

        def storeA(i, slot):
            val = stagA[slot].astype(jnp.bfloat16)
            if i == 0:
                wq_ref[...] = val
            elif i in (1, 2):
                x_ref[pl.ds((i - 1) * CH, CH), :] = val
            else:
                wo_ref[...] = val

        dmasA = [None, None]
        dmasB = [None, None]

        def startA(i):
            d = pltpu.make_async_copy(srcsA[i], stagA.at[i % 2],
                                      copyA_sems.at[i % 2])
            d.start()
            dmasA[i % 2] = d

        def finishA(i):
            dmasA[i % 2].wait()
            storeA(i, i % 2)

        def startB(b):
            hbm = k_hbm if b < HL else v_hbm
            d = pltpu.make_async_copy(hbm.at[:, b % HL, :],
                                      stagB.at[b % 2],
                                      copyB_sems.at[b % 2])
            d.start()
            dmasB[b % 2] = d

        def finishB(b):
            dmasB[b % 2].wait()
            dst = k_ref if b < HL else v_ref
            dst[b % HL] = stagB[b % 2].astype(jnp.bfloat16)

        startA(0)
        startA(1)
        pl.semaphore_wait(barrier_sem, 3)
        finishA(0)
        startA(2)
        finishA(1)
        startB(0)
        startB(1)
        for j8 in range(SQ // QB):
            if j8 == 2:
                finishA(2)
                startA(3)
            rows = pl.ds(j8 * QB, QB)
            q_ref[rows, :] = (lax.dot_general(
                x_ref[rows, :], wq_ref[...], (((1,), (0,)), ((), ())),
                preferred_element_type=jnp.float32)
                * QSCALE).astype(jnp.bfloat16)
            for t in range(2):
                b = 2 * j8 + t
                finishB(b)
                if b + 2 < 2 * HL:
                    startB(b + 2)
        finishA(3)

        def compute_block(qtr, j):
            if True:
                row = qtr * QROWS + j * QB
                rows = pl.ds(row, QB)
                s = jnp.minimum(jnp.maximum(row - 128, 0), SKV - KW)
                s = (s // 128) * 128
                qi = lax.broadcasted_iota(jnp.int32, (QB, KW), 0) + row
                kj = lax.broadcasted_iota(jnp.int32, (QB, KW), 1) + s
                maskbias = jnp.where(jnp.abs(qi - kj) <= WIN, 0.0, -1e30)
                for h in range(HL):
                    hcols = pl.ds(h * DH, DH)
                    qblk = q_ref[rows, hcols]
                    kwin = k_ref[h, pl.ds(s, KW), :]
                    vwin = v_ref[h, pl.ds(s, KW), :]
                    scores = lax.dot_general(
                        qblk, kwin, (((1,), (1,)), ((), ())),
                        preferred_element_type=jnp.float32)
                    w = jnp.exp2(scores + maskbias)
                    recip = 1.0 / jnp.sum(w, axis=1, keepdims=True)
                    ctx_blk = lax.dot_general(
                        w.astype(jnp.bfloat16), vwin, (((1,), (0,)), ((), ())),
                        preferred_element_type=jnp.float32) * recip
                    ctx_ref[rows, hcols] = ctx_blk.astype(jnp.bfloat16)
                out_ref[0, rows, :] = lax.dot_general(
                    ctx_ref[rows, :], wo_ref[...], (((1,), (0,)), ((), ())),
                    preferred_element_type=jnp.float32)

        def qmod(c):
            return lax.rem(my_pos + c + 2 * N_DEV, N_DEV)

        def rowQ(q):
            return pl.ds(q * QROWS, QROWS)

        def rowA(q):
            return pl.ds(q * QROWS, QB)

        def rowB(q):
            return pl.ds(q * QROWS + QB, QB)

        dr_rdmas = []
        for r in (1, 2, 3):
            qtr = qmod(r)
            compute_block(qtr, 0)
            compute_block(qtr, 1)
            dr_send[r - 1] = out_ref[0, rowQ(qtr), :].astype(jnp.bfloat16)
            rdma = pltpu.make_async_remote_copy(
                src_ref=dr_send.at[r - 1],
                dst_ref=dr_recv.at[3 - r],
                send_sem=drs_sems.at[r - 1],
                recv_sem=drr_sems.at[3 - r],
                device_id=(qmod(r),),
                device_id_type=pl.DeviceIdType.MESH,
            )
            rdma.start()
            dr_rdmas.append(rdma)

        def copy(src, dst, s_sems, r_sems, idx, dev):
            return pltpu.make_async_remote_copy(
                src_ref=src, dst_ref=dst,
                send_sem=s_sems.at[idx], recv_sem=r_sems.at[idx],
                device_id=(dev,), device_id_type=pl.DeviceIdType.MESH)

        started = []
        compute_block(qmod(0), 0)
        for s in range(3):
            pltpu.make_async_remote_copy(
                src_ref=dr_send.at[s], dst_ref=dr_recv.at[s],
                send_sem=drs_sems.at[s], recv_sem=drr_sems.at[s],
                device_id=(right,), device_id_type=pl.DeviceIdType.MESH,
            ).wait_recv()
        for j in range(2):
            if j == 1:
                compute_block(qmod(0), 1)
            rows = pl.ds(qmod(0) * QROWS + j * QB, QB)
            srows = pl.ds(j * QB, QB)
            acc = out_ref[0, rows, :]
            for s in range(3):
                acc = acc + dr_recv[s, srows, :].astype(jnp.float32)
            out_ref[0, rows, :] = acc
            ag_send = ag_sendR if j == 0 else ag_sendL
            for u in range(2):
                ag_send[u] = acc[u * HQB:(u + 1) * HQB, :].astype(jnp.bfloat16)
            for u in range(2):
                if j == 0:
                    d = copy(ag_sendR.at[u], ag_recvR.at[0, u],
                             sendR_sems, recvR_sems, u, right)
                else:
                    d = copy(ag_sendL.at[u], ag_recvL.at[0, u],
                             sendL_sems, recvL_sems, u, left)
                d.start()
                started.append(d)
        for rdma in dr_rdmas:
            rdma.wait_send()

        for t in range(N_DEV - 1):
            qR = qmod(-1 - t)
            qL = qmod(1 + t)
            for u in range(2):
                copy(ag_sendR.at[u], ag_recvR.at[t, u],
                     sendR_sems, recvR_sems, 2 * t + u, right).wait_recv()
                copy(ag_sendL.at[u], ag_recvL.at[t, u],
                     sendL_sems, recvL_sems, 2 * t + u, left).wait_recv()
                if t < N_DEV - 2:
                    fR = copy(ag_recvR.at[t, u], ag_recvR.at[t + 1, u],
                              sendR_sems, recvR_sems, 2 * (t + 1) + u, right)
                    fL = copy(ag_recvL.at[t, u], ag_recvL.at[t + 1, u],
                              sendL_sems, recvL_sems, 2 * (t + 1) + u, left)
                    fR.start()
                    fL.start()
                    started += [fR, fL]
                out_ref[0, pl.ds(qR * QROWS + u * HQB, HQB), :] = (
                    ag_recvR[t, u].astype(jnp.float32))
                out_ref[0, pl.ds(qL * QROWS + QB + u * HQB, HQB), :] = (
                    ag_recvL[t, u].astype(jnp.float32))
        for d in started:
            d.wait_send()

        @functools.partial(pl.run_scoped, sem2=pltpu.SemaphoreType.REGULAR)
        def _(sem2):
            for nbr in (left, right, diag):
                pl.semaphore_signal(sem2, inc=1, device_id=(nbr,),
                                    device_id_type=pl.DeviceIdType.MESH)
            pl.semaphore_wait(sem2, 3)

    return pl.pallas_call(
        body,
        out_shape=jax.ShapeDtypeStruct((1, SQ, DM), jnp.float32),
        in_specs=[pl.BlockSpec(memory_space=pl.ANY)] * 5,
        out_specs=pl.BlockSpec(memory_space=pltpu.VMEM),
        scratch_shapes=[
            pltpu.VMEM((2, 1024, DM), jnp.float32),
            pltpu.VMEM((2, SKV, DH), jnp.float32),
            pltpu.VMEM((SQ, DM), jnp.bfloat16),
            pltpu.VMEM((HL, SKV, DH), jnp.bfloat16),
            pltpu.VMEM((HL, SKV, DH), jnp.bfloat16),
            pltpu.VMEM((DM, DL), jnp.bfloat16),
            pltpu.VMEM((DL, DM), jnp.bfloat16),
            pltpu.VMEM((SQ, DL), jnp.bfloat16),
            pltpu.VMEM((SQ, DL), jnp.bfloat16),
            pltpu.VMEM((3, QROWS, DM), jnp.bfloat16),
            pltpu.VMEM((3, QROWS, DM), jnp.bfloat16),
            pltpu.VMEM((2, HQB, DM), jnp.bfloat16),
            pltpu.VMEM((3, 2, HQB, DM), jnp.bfloat16),
            pltpu.VMEM((2, HQB, DM), jnp.bfloat16),
            pltpu.VMEM((3, 2, HQB, DM), jnp.bfloat16),
            pltpu.SemaphoreType.DMA((2,)),
            pltpu.SemaphoreType.DMA((2,)),
            pltpu.SemaphoreType.DMA((3,)),
            pltpu.SemaphoreType.DMA((3,)),
            pltpu.SemaphoreType.DMA((6,)),
            pltpu.SemaphoreType.DMA((6,)),
            pltpu.SemaphoreType.DMA((6,)),
            pltpu.SemaphoreType.DMA((6,)),
        ],
        compiler_params=pltpu.CompilerParams(
            collective_id=0,
            vmem_limit_bytes=100 * 1024 * 1024,
        ),
    )(xb, Wq, kb, vb, Wo)


# device time: 89489 ns/iter; 1.0319x vs baseline; 1.0319x over previous
import functools

import jax
import jax.numpy as jnp
from jax import lax
from jax.experimental import pallas as pl
from jax.experimental.pallas import tpu as pltpu

N_DEV = 4
SQ = 2048
SKV = 2048
DM = 1024
HL = 8
DH = 128
DL = HL * DH
QB = 256
KW = 512
WIN = 128
QROWS = SQ // N_DEV
SCALE = 0.08838834764831843
QSCALE = SCALE * 1.4426950408889634
HQB = 128


def kernel(x, Wq, K_ext, V_ext, Wo):
    xb = x[0]
    kb = K_ext[0]
    vb = V_ext[0]

    def body(x_hbm, wq_hbm, k_hbm, v_hbm, wo_hbm, out_ref,
             stagA, stagB, x_ref, k_ref, v_ref, wq_ref, wo_ref,
             q_ref, ctx_ref,
             dr_send, dr_recv, ag_sendR, ag_recvR, ag_sendL, ag_recvL,
             copyA_sems, copyB_sems, drs_sems, drr_sems,
             sendR_sems, recvR_sems, sendL_sems, recvL_sems):
        my_pos = lax.axis_index("i")
        left = lax.rem(my_pos + N_DEV - 1, N_DEV)
        right = lax.rem(my_pos + 1, N_DEV)
        diag = lax.rem(my_pos + 2, N_DEV)

        barrier_sem = pltpu.get_barrier_semaphore()
        for nbr in (left, right, diag):
            pl.semaphore_signal(barrier_sem, inc=1, device_id=(nbr,),
                                device_id_type=pl.DeviceIdType.MESH)

        CH = 1024
        srcsA = [
            wq_hbm.at[:, pl.ds(my_pos * DL, DL)],
            x_hbm.at[pl.ds(0, CH), :],
            x_hbm.at[pl.ds(CH, CH), :],
            wo_hbm.at[pl.ds(my_pos * DL, DL), :],
        ]

        def storeA(i, slot):
            val = stagA[slot].astype(jnp.bfloat16)
            if i == 0:
                wq_ref[...] = val
            elif i in (1, 2):
                x_ref[pl.ds((i - 1) * CH, CH), :] = val
            else:
                wo_ref[...] = val

        dmasA = [None, None]
        dmasB = [None, None]

        def startA(i):
            d = pltpu.make_async_copy(srcsA[i], stagA.at[i % 2],
                                      copyA_sems.at[i % 2])
            d.start()
            dmasA[i % 2] = d

        def finishA(i):
            dmasA[i % 2].wait()
            storeA(i, i % 2)

        def startB(b):
            hbm = k_hbm if b < HL else v_hbm
            d = pltpu.make_async_copy(hbm.at[:, b % HL, :],
                                      stagB.at[b % 2],
                                      copyB_sems.at[b % 2])
            d.start()
            dmasB[b % 2] = d

        def finishB(b):
            dmasB[b % 2].wait()
            dst = k_ref if b < HL else v_ref
            dst[b % HL] = stagB[b % 2].astype(jnp.bfloat16)

        startA(0)
        startA(1)
        pl.semaphore_wait(barrier_sem, 3)
        finishA(0)
        startA(2)
        finishA(1)
        startB(0)
        startB(1)
        for j8 in range(SQ // QB):
            if j8 == 2:
                finishA(2)
                startA(3)
            rows = pl.ds(j8 * QB, QB)
            q_ref[rows, :] = (lax.dot_general(
                x_ref[rows, :], wq_ref[...], (((1,), (0,)), ((), ())),
                preferred_element_type=jnp.float32)
                * QSCALE).astype(jnp.bfloat16)
            for t in range(2):
                b = 2 * j8 + t
                finishB(b)
                if b + 2 < 2 * HL:
                    startB(b + 2)
        finishA(3)

        def compute_block(qtr, j):
            if True:
                row = qtr * QROWS + j * QB
                rows = pl.ds(row, QB)
                s = jnp.minimum(jnp.maximum(row - 128, 0), SKV - KW)
                s = (s // 128) * 128
                qi = lax.broadcasted_iota(jnp.int32, (QB, KW), 0) + row
                kj = lax.broadcasted_iota(jnp.int32, (QB, KW), 1) + s
                maskbias = jnp.where(jnp.abs(qi - kj) <= WIN, 0.0, -1e30)
                for h in range(HL):
                    hcols = pl.ds(h * DH, DH)
                    qblk = q_ref[rows, hcols]
                    kwin = k_ref[h, pl.ds(s, KW), :]
                    vwin = v_ref[h, pl.ds(s, KW), :]
                    scores = lax.dot_general(
                        qblk, kwin, (((1,), (1,)), ((), ())),
                        preferred_element_type=jnp.float32)
                    w = jnp.exp2(scores + maskbias)
                    recip = 1.0 / jnp.sum(w, axis=1, keepdims=True)
                    ctx_blk = lax.dot_general(
                        w.astype(jnp.bfloat16), vwin, (((1,), (0,)), ((), ())),
                        preferred_element_type=jnp.float32) * recip
                    ctx_ref[rows, hcols] = ctx_blk.astype(jnp.bfloat16)
                out_ref[0, rows, :] = lax.dot_general(
                    ctx_ref[rows, :], wo_ref[...], (((1,), (0,)), ((), ())),
                    preferred_element_type=jnp.float32)

        def qmod(c):
            return lax.rem(my_pos + c + 2 * N_DEV, N_DEV)

        def rowQ(q):
            return pl.ds(q * QROWS, QROWS)

        def rowA(q):
            return pl.ds(q * QROWS, QB)

        def rowB(q):
            return pl.ds(q * QROWS + QB, QB)

        dr_rdmas = []
        for r in (1, 2, 3):
            qtr = qmod(r)
            compute_block(qtr, 0)
            compute_block(qtr, 1)
            dr_send[r - 1] = out_ref[0, rowQ(qtr), :].astype(jnp.bfloat16)
            rdma = pltpu.make_async_remote_copy(
                src_ref=dr_send.at[r - 1],
                dst_ref=dr_recv.at[3 - r],
                send_sem=drs_sems.at[r - 1],
                recv_sem=drr_sems.at[3 - r],
                device_id=(qmod(r),),
                device_id_type=pl.DeviceIdType.MESH,
            )
            rdma.start()
            dr_rdmas.append(rdma)

        def copy(src, dst, s_sems, r_sems, idx, dev):
            return pltpu.make_async_remote_copy(
                src_ref=src, dst_ref=dst,
                send_sem=s_sems.at[idx], recv_sem=r_sems.at[idx],
                device_id=(dev,), device_id_type=pl.DeviceIdType.MESH)

        started = []
        compute_block(qmod(0), 0)
        compute_block(qmod(0), 1)
        for s in range(3):
            pltpu.make_async_remote_copy(
                src_ref=dr_send.at[s], dst_ref=dr_recv.at[s],
                send_sem=drs_sems.at[s], recv_sem=drr_sems.at[s],
                device_id=(right,), device_id_type=pl.DeviceIdType.MESH,
            ).wait_recv()
        for j in range(2):
            rows = pl.ds(qmod(0) * QROWS + j * QB, QB)
            srows = pl.ds(j * QB, QB)
            acc = out_ref[0, rows, :]
            for s in range(3):
                acc = acc + dr_recv[s, srows, :].astype(jnp.float32)
            out_ref[0, rows, :] = acc
            ag_send = ag_sendR if j == 0 else ag_sendL
            for u in range(2):
                ag_send[u] = acc[u * HQB:(u + 1) * HQB, :].astype(jnp.bfloat16)
            for u in range(2):
                if j == 0:
                    d = copy(ag_sendR.at[u], ag_recvR.at[0, u],
                             sendR_sems, recvR_sems, u, right)
                else:
                    d = copy(ag_sendL.at[u], ag_recvL.at[0, u],
                             sendL_sems, recvL_sems, u, left)
                d.start()
                started.append(d)
        for rdma in dr_rdmas:
            rdma.wait_send()

        for t in range(N_DEV - 1):
            qR = qmod(-1 - t)
            qL = qmod(1 + t)
            for u in range(2):
                copy(ag_sendR.at[u], ag_recvR.at[t, u],
                     sendR_sems, recvR_sems, 2 * t + u, right).wait_recv()
                copy(ag_sendL.at[u], ag_recvL.at[t, u],
                     sendL_sems, recvL_sems, 2 * t + u, left).wait_recv()
                if t < N_DEV - 2:
                    fR = copy(ag_recvR.at[t, u], ag_recvR.at[t + 1, u],
                              sendR_sems, recvR_sems, 2 * (t + 1) + u, right)
                    fL = copy(ag_recvL.at[t, u], ag_recvL.at[t + 1, u],
                              sendL_sems, recvL_sems, 2 * (t + 1) + u, left)
                    fR.start()
                    fL.start()
                    started += [fR, fL]
                out_ref[0, pl.ds(qR * QROWS + u * HQB, HQB), :] = (
                    ag_recvR[t, u].astype(jnp.float32))
                out_ref[0, pl.ds(qL * QROWS + QB + u * HQB, HQB), :] = (
                    ag_recvL[t, u].astype(jnp.float32))
        for d in started:
            d.wait_send()

        @functools.partial(pl.run_scoped, sem2=pltpu.SemaphoreType.REGULAR)
        def _(sem2):
            for nbr in (left, right, diag):
                pl.semaphore_signal(sem2, inc=1, device_id=(nbr,),
                                    device_id_type=pl.DeviceIdType.MESH)
            pl.semaphore_wait(sem2, 3)

    return pl.pallas_call(
        body,
        out_shape=jax.ShapeDtypeStruct((1, SQ, DM), jnp.float32),
        in_specs=[pl.BlockSpec(memory_space=pl.ANY)] * 5,
        out_specs=pl.BlockSpec(memory_space=pltpu.VMEM),
        scratch_shapes=[
            pltpu.VMEM((2, 1024, DM), jnp.float32),
            pltpu.VMEM((2, SKV, DH), jnp.float32),
            pltpu.VMEM((SQ, DM), jnp.bfloat16),
            pltpu.VMEM((HL, SKV, DH), jnp.bfloat16),
            pltpu.VMEM((HL, SKV, DH), jnp.bfloat16),
            pltpu.VMEM((DM, DL), jnp.bfloat16),
            pltpu.VMEM((DL, DM), jnp.bfloat16),
            pltpu.VMEM((SQ, DL), jnp.bfloat16),
            pltpu.VMEM((SQ, DL), jnp.bfloat16),
            pltpu.VMEM((3, QROWS, DM), jnp.bfloat16),
            pltpu.VMEM((3, QROWS, DM), jnp.bfloat16),
            pltpu.VMEM((2, HQB, DM), jnp.bfloat16),
            pltpu.VMEM((3, 2, HQB, DM), jnp.bfloat16),
            pltpu.VMEM((2, HQB, DM), jnp.bfloat16),
            pltpu.VMEM((3, 2, HQB, DM), jnp.bfloat16),
            pltpu.SemaphoreType.DMA((2,)),
            pltpu.SemaphoreType.DMA((2,)),
            pltpu.SemaphoreType.DMA((3,)),
            pltpu.SemaphoreType.DMA((3,)),
            pltpu.SemaphoreType.DMA((6,)),
            pltpu.SemaphoreType.DMA((6,)),
            pltpu.SemaphoreType.DMA((6,)),
            pltpu.SemaphoreType.DMA((6,)),
        ],
        compiler_params=pltpu.CompilerParams(
            collective_id=0,
            vmem_limit_bytes=100 * 1024 * 1024,
        ),
    )(xb, Wq, kb, vb, Wo)
